# baseline (speedup 1.0000x reference)
import jax
import jax.numpy as jnp
from jax.experimental import pallas as pl

INPUT_DIM = 512
DIM = 256
K = 1024
M_BOOK = 4
BATCH = 1024
BN_EPS = 1e-5

BLK = 128

_HI = jax.lax.Precision.HIGHEST


def _nt(a, b):
    return jax.lax.dot_general(a, b, (((1,), (1,)), ((), ())),
                               preferred_element_type=jnp.float32)


def _bn_relu(h, g, b):
    return jax.nn.relu(h / jnp.sqrt(jnp.float32(1.0 + BN_EPS)) * g + b)


def _rowsq(v):
    s = v * v
    acc = s[:, 0:8]
    for r in range(1, 32):
        acc = acc + s[:, 8 * r:8 * r + 8]
    t = acc[:, 0:4] + acc[:, 4:8]
    t = t[:, 0:2] + t[:, 2:4]
    return t[:, 0:1] + t[:, 1:2]


def _colsq_fold(e):
    s = e * e
    w = DIM
    while w > 1:
        w //= 2
        s = s[:, :w] + s[:, w:]
    return s


def _body(x_ref, mask_ref, pos_ref,
          eW1, eb1, eg1, ebe1, eW2, eb2, eg2, ebe2, eW3, eb3,
          dW1, db1, dg1, dbe1, dW2, db2, dg2, dbe2, dW3, db3,
          cb_ref,
          xhat_ref, r0, r1, r2, r3, c0, c1, c2, c3):
    pid = pl.program_id(0)
    res_refs = (r0, r1, r2, r3)
    ce_refs = (c0, c1, c2, c3)

    xb = x_ref[...]
    maskf = mask_ref[...]
    xb = jnp.where(maskf > 0.5, 0.0, xb)
    row = jax.lax.broadcasted_iota(jnp.int32, (BLK, 1), 0) + pid * BLK
    xb = jnp.where(row == 0, 0.0, xb)
    xb = xb + pos_ref[...]

    h = _nt(xb, eW1[...]) + eb1[...]
    h = _bn_relu(h, eg1[...], ebe1[...])
    h = _nt(h, eW2[...]) + eb2[...]
    h = _bn_relu(h, eg2[...], ebe2[...])
    ze = _nt(h, eW3[...]) + eb3[...]

    iota = jax.lax.broadcasted_iota(jnp.int32, (BLK, K), 1)
    eye = (jax.lax.broadcasted_iota(jnp.int32, (K, K), 0)
           == jax.lax.broadcasted_iota(jnp.int32, (K, K), 1)
           ).astype(jnp.float32)
    cur = ze
    zq = None
    for m in range(M_BOOK):
        emb = cb_ref[m]
        rowsq = _rowsq(cur)
        embsq_col = _colsq_fold(emb)
        embsq = jax.lax.dot_general(embsq_col, eye,
                                    (((0,), (0,)), ((), ())),
                                    preferred_element_type=jnp.float32,
                                    precision=_HI)
        score = _nt(cur, emb)
        dist = (rowsq + embsq) - 2.0 * score
        dmin = jnp.min(dist, axis=1, keepdims=True)
        nearest = jnp.min(jnp.where(dist == dmin, iota, K),
                          axis=1, keepdims=True)
        onehot = (iota == nearest).astype(jnp.float32)
        ce = jax.lax.dot_general(onehot, emb, (((1,), (0,)), ((), ())),
                                 preferred_element_type=jnp.float32,
                                 precision=_HI)
        res_refs[m][...] = cur
        ce_refs[m][...] = ce
        zq = ce if zq is None else zq + ce
        cur = cur - ce

    di = ze + (zq - ze)
    h = _nt(di, dW1[...]) + db1[...]
    h = _bn_relu(h, dg1[...], dbe1[...])
    h = _nt(h, dW2[...]) + db2[...]
    h = _bn_relu(h, dg2[...], dbe2[...])
    xhat_ref[...] = _nt(h, dW3[...]) + db3[...]


def kernel(x, enc_W1, enc_b1, enc_g1, enc_be1, enc_W2, enc_b2, enc_g2,
           enc_be2, enc_W3, enc_b3, dec_W1, dec_b1, dec_g1, dec_be1,
           dec_W2, dec_b2, dec_g2, dec_be2, dec_W3, dec_b3,
           codebooks, pos, mask):
    row = lambda v: v.reshape(1, -1).astype(jnp.float32)
    maskf = row(mask)

    grid = BATCH // BLK
    full = lambda a: pl.BlockSpec(a.shape, lambda i: (0,) * a.ndim)
    batch_spec = lambda w: pl.BlockSpec((BLK, w), lambda i: (i, 0))

    args = (x, maskf, pos,
            enc_W1, row(enc_b1), row(enc_g1), row(enc_be1),
            enc_W2, row(enc_b2), row(enc_g2), row(enc_be2),
            enc_W3, row(enc_b3),
            dec_W1, row(dec_b1), row(dec_g1), row(dec_be1),
            dec_W2, row(dec_b2), row(dec_g2), row(dec_be2),
            dec_W3, row(dec_b3),
            codebooks)
    in_specs = [batch_spec(INPUT_DIM)] + [full(a) for a in args[1:]]

    out_shapes = ([jax.ShapeDtypeStruct((BATCH, INPUT_DIM), jnp.float32)]
                  + [jax.ShapeDtypeStruct((BATCH, DIM), jnp.float32)] * 8)
    out_specs = ([batch_spec(INPUT_DIM)] + [batch_spec(DIM)] * 8)

    outs = pl.pallas_call(
        _body,
        grid=(grid,),
        in_specs=in_specs,
        out_specs=out_specs,
        out_shape=out_shapes,
    )(*args)

    x_hat = outs[0]
    res_list = tuple(outs[1:5])
    ce_list = tuple(outs[5:9])
    return (x_hat, res_list, ce_list)


# speedup vs baseline: 1.6826x; 1.6826x over previous
import jax
import jax.numpy as jnp
from jax.experimental import pallas as pl

INPUT_DIM = 512
DIM = 256
K = 1024
M_BOOK = 4
BATCH = 1024
BN_EPS = 1e-5

BLK = 256

_HI = jax.lax.Precision.HIGHEST


def _nt(a, b):
    return jax.lax.dot_general(a, b, (((1,), (1,)), ((), ())),
                               preferred_element_type=jnp.float32)


def _bn_relu(h, g, b):
    return jax.nn.relu(h / jnp.sqrt(jnp.float32(1.0 + BN_EPS)) * g + b)


def _rowsq(v):
    s = v * v
    acc = s[:, 0:8]
    for r in range(1, 32):
        acc = acc + s[:, 8 * r:8 * r + 8]
    t = acc[:, 0:4] + acc[:, 4:8]
    t = t[:, 0:2] + t[:, 2:4]
    return t[:, 0:1] + t[:, 1:2]


def _rowsq_t(et):
    s = et * et
    w = DIM
    while w > 1:
        w //= 2
        s = s[:w, :] + s[w:, :]
    return s


def _body(x_ref, mask_ref, pos_ref,
          eW1, eb1, eg1, ebe1, eW2, eb2, eg2, ebe2, eW3, eb3,
          dW1, db1, dg1, dbe1, dW2, db2, dg2, dbe2, dW3, db3,
          cb_ref, cbt_ref,
          xhat_ref, r0, r1, r2, r3, c0, c1, c2, c3):
    pid = pl.program_id(0)
    res_refs = (r0, r1, r2, r3)
    ce_refs = (c0, c1, c2, c3)

    xb = x_ref[...]
    maskf = mask_ref[...]
    xb = jnp.where(maskf > 0.5, 0.0, xb)
    row = jax.lax.broadcasted_iota(jnp.int32, (BLK, 1), 0) + pid * BLK
    xb = jnp.where(row == 0, 0.0, xb)
    xb = xb + pos_ref[...]

    h = _nt(xb, eW1[...]) + eb1[...]
    h = _bn_relu(h, eg1[...], ebe1[...])
    h = _nt(h, eW2[...]) + eb2[...]
    h = _bn_relu(h, eg2[...], ebe2[...])
    ze = _nt(h, eW3[...]) + eb3[...]

    iota = jax.lax.broadcasted_iota(jnp.int32, (BLK, K), 1)
    cur = ze
    zq = None
    for m in range(M_BOOK):
        emb = cb_ref[m]
        rowsq = _rowsq(cur)
        embsq = _rowsq_t(cbt_ref[m])
        score = _nt(cur, emb)
        dist = (rowsq + embsq) - 2.0 * score
        dmin = jnp.min(dist, axis=1, keepdims=True)
        nearest = jnp.min(jnp.where(dist == dmin, iota, K),
                          axis=1, keepdims=True)
        onehot = (iota == nearest).astype(jnp.float32)
        ce = jax.lax.dot_general(onehot, emb, (((1,), (0,)), ((), ())),
                                 preferred_element_type=jnp.float32,
                                 precision=_HI)
        res_refs[m][...] = cur
        ce_refs[m][...] = ce
        zq = ce if zq is None else zq + ce
        cur = cur - ce

    di = ze + (zq - ze)
    h = _nt(di, dW1[...]) + db1[...]
    h = _bn_relu(h, dg1[...], dbe1[...])
    h = _nt(h, dW2[...]) + db2[...]
    h = _bn_relu(h, dg2[...], dbe2[...])
    xhat_ref[...] = _nt(h, dW3[...]) + db3[...]


def kernel(x, enc_W1, enc_b1, enc_g1, enc_be1, enc_W2, enc_b2, enc_g2,
           enc_be2, enc_W3, enc_b3, dec_W1, dec_b1, dec_g1, dec_be1,
           dec_W2, dec_b2, dec_g2, dec_be2, dec_W3, dec_b3,
           codebooks, pos, mask):
    row = lambda v: v.reshape(1, -1).astype(jnp.float32)
    maskf = row(mask)

    grid = BATCH // BLK
    full = lambda a: pl.BlockSpec(a.shape, lambda i: (0,) * a.ndim)
    batch_spec = lambda w: pl.BlockSpec((BLK, w), lambda i: (i, 0))

    args = (x, maskf, pos,
            enc_W1, row(enc_b1), row(enc_g1), row(enc_be1),
            enc_W2, row(enc_b2), row(enc_g2), row(enc_be2),
            enc_W3, row(enc_b3),
            dec_W1, row(dec_b1), row(dec_g1), row(dec_be1),
            dec_W2, row(dec_b2), row(dec_g2), row(dec_be2),
            dec_W3, row(dec_b3),
            codebooks, jnp.transpose(codebooks, (0, 2, 1)))
    in_specs = [batch_spec(INPUT_DIM)] + [full(a) for a in args[1:]]

    out_shapes = ([jax.ShapeDtypeStruct((BATCH, INPUT_DIM), jnp.float32)]
                  + [jax.ShapeDtypeStruct((BATCH, DIM), jnp.float32)] * 8)
    out_specs = ([batch_spec(INPUT_DIM)] + [batch_spec(DIM)] * 8)

    outs = pl.pallas_call(
        _body,
        grid=(grid,),
        in_specs=in_specs,
        out_specs=out_specs,
        out_shape=out_shapes,
    )(*args)

    x_hat = outs[0]
    res_list = tuple(outs[1:5])
    ce_list = tuple(outs[5:9])
    return (x_hat, res_list, ce_list)


# speedup vs baseline: 1.7301x; 1.0282x over previous
import jax
import jax.numpy as jnp
from jax.experimental import pallas as pl
from jax.experimental.pallas import tpu as pltpu

INPUT_DIM = 512
DIM = 256
K = 1024
M_BOOK = 4
BATCH = 1024
BN_EPS = 1e-5

BLK = 256

_HI = jax.lax.Precision.HIGHEST


def _nt(a, b):
    return jax.lax.dot_general(a, b, (((1,), (1,)), ((), ())),
                               preferred_element_type=jnp.float32)


def _bn_relu(h, g, b):
    return jax.nn.relu(h / jnp.sqrt(jnp.float32(1.0 + BN_EPS)) * g + b)


def _rowsq(v):
    s = v * v
    acc = s[:, 0:8]
    for r in range(1, 32):
        acc = acc + s[:, 8 * r:8 * r + 8]
    t = acc[:, 0:4] + acc[:, 4:8]
    t = t[:, 0:2] + t[:, 2:4]
    return t[:, 0:1] + t[:, 1:2]


def _colsq_fold(e):
    s = e * e
    w = DIM
    while w > 1:
        w //= 2
        s = s[:, :w] + s[:, w:]
    return s


def _body(x_ref, mask_ref, pos_ref,
          eW1, eb1, eg1, ebe1, eW2, eb2, eg2, ebe2, eW3, eb3,
          dW1, db1, dg1, dbe1, dW2, db2, dg2, dbe2, dW3, db3,
          cb_ref,
          xhat_ref, r0, r1, r2, r3, c0, c1, c2, c3,
          es_ref):
    pid = pl.program_id(0)
    res_refs = (r0, r1, r2, r3)
    ce_refs = (c0, c1, c2, c3)

    xb = x_ref[...]
    maskf = mask_ref[...]
    xb = jnp.where(maskf > 0.5, 0.0, xb)
    row = jax.lax.broadcasted_iota(jnp.int32, (BLK, 1), 0) + pid * BLK
    xb = jnp.where(row == 0, 0.0, xb)
    xb = xb + pos_ref[...]

    h = _nt(xb, eW1[...]) + eb1[...]
    h = _bn_relu(h, eg1[...], ebe1[...])
    h = _nt(h, eW2[...]) + eb2[...]
    h = _bn_relu(h, eg2[...], ebe2[...])
    ze = _nt(h, eW3[...]) + eb3[...]

    @pl.when(pid == 0)
    def _():
        eye = (jax.lax.broadcasted_iota(jnp.int32, (K, K), 0)
               == jax.lax.broadcasted_iota(jnp.int32, (K, K), 1)
               ).astype(jnp.float32)
        for m in range(M_BOOK):
            embsq_col = _colsq_fold(cb_ref[m])
            es_ref[m:m + 1, :] = jax.lax.dot_general(
                embsq_col, eye, (((0,), (0,)), ((), ())),
                preferred_element_type=jnp.float32, precision=_HI)

    iota = jax.lax.broadcasted_iota(jnp.int32, (BLK, K), 1)
    cur = ze
    zq = None
    for m in range(M_BOOK):
        emb = cb_ref[m]
        rowsq = _rowsq(cur)
        embsq = es_ref[m:m + 1, :]
        score = _nt(cur, emb)
        dist = (rowsq + embsq) - 2.0 * score
        dmin = jnp.min(dist, axis=1, keepdims=True)
        nearest = jnp.min(jnp.where(dist == dmin, iota, K),
                          axis=1, keepdims=True)
        onehot = (iota == nearest).astype(jnp.float32)
        ce = jax.lax.dot_general(onehot, emb, (((1,), (0,)), ((), ())),
                                 preferred_element_type=jnp.float32,
                                 precision=_HI)
        res_refs[m][...] = cur
        ce_refs[m][...] = ce
        zq = ce if zq is None else zq + ce
        cur = cur - ce

    di = ze + (zq - ze)
    h = _nt(di, dW1[...]) + db1[...]
    h = _bn_relu(h, dg1[...], dbe1[...])
    h = _nt(h, dW2[...]) + db2[...]
    h = _bn_relu(h, dg2[...], dbe2[...])
    xhat_ref[...] = _nt(h, dW3[...]) + db3[...]


def kernel(x, enc_W1, enc_b1, enc_g1, enc_be1, enc_W2, enc_b2, enc_g2,
           enc_be2, enc_W3, enc_b3, dec_W1, dec_b1, dec_g1, dec_be1,
           dec_W2, dec_b2, dec_g2, dec_be2, dec_W3, dec_b3,
           codebooks, pos, mask):
    row = lambda v: v.reshape(1, -1).astype(jnp.float32)
    maskf = row(mask)

    grid = BATCH // BLK
    full = lambda a: pl.BlockSpec(a.shape, lambda i: (0,) * a.ndim)
    batch_spec = lambda w: pl.BlockSpec((BLK, w), lambda i: (i, 0))

    args = (x, maskf, pos,
            enc_W1, row(enc_b1), row(enc_g1), row(enc_be1),
            enc_W2, row(enc_b2), row(enc_g2), row(enc_be2),
            enc_W3, row(enc_b3),
            dec_W1, row(dec_b1), row(dec_g1), row(dec_be1),
            dec_W2, row(dec_b2), row(dec_g2), row(dec_be2),
            dec_W3, row(dec_b3),
            codebooks)
    in_specs = [batch_spec(INPUT_DIM)] + [full(a) for a in args[1:]]

    out_shapes = ([jax.ShapeDtypeStruct((BATCH, INPUT_DIM), jnp.float32)]
                  + [jax.ShapeDtypeStruct((BATCH, DIM), jnp.float32)] * 8)
    out_specs = ([batch_spec(INPUT_DIM)] + [batch_spec(DIM)] * 8)

    outs = pl.pallas_call(
        _body,
        grid=(grid,),
        in_specs=in_specs,
        out_specs=out_specs,
        out_shape=out_shapes,
        scratch_shapes=[pltpu.VMEM((8, K), jnp.float32)],
    )(*args)

    x_hat = outs[0]
    res_list = tuple(outs[1:5])
    ce_list = tuple(outs[5:9])
    return (x_hat, res_list, ce_list)


# speedup vs baseline: 2.0913x; 1.2087x over previous
import jax
import jax.numpy as jnp
from jax.experimental import pallas as pl
from jax.experimental.pallas import tpu as pltpu

INPUT_DIM = 512
DIM = 256
K = 1024
M_BOOK = 4
BATCH = 1024
BN_EPS = 1e-5

BLK = 256

_HI = jax.lax.Precision.HIGHEST


def _nt(a, b):
    return jax.lax.dot_general(a, b, (((1,), (1,)), ((), ())),
                               preferred_element_type=jnp.float32)


def _bn_relu(h, g, b):
    return jax.nn.relu(h / jnp.sqrt(jnp.float32(1.0 + BN_EPS)) * g + b)


def _rowsq(v):
    s = v * v
    acc = s[:, 0:8]
    for r in range(1, 32):
        acc = acc + s[:, 8 * r:8 * r + 8]
    t = acc[:, 0:4] + acc[:, 4:8]
    t = t[:, 0:2] + t[:, 2:4]
    return t[:, 0:1] + t[:, 1:2]


def _colsq_fold(e):
    s = e * e
    w = DIM
    while w > 1:
        w //= 2
        s = s[:, :w] + s[:, w:]
    return s


def _body(x_ref, mask_ref, pos_ref,
          eW1, eb1, eg1, ebe1, eW2, eb2, eg2, ebe2, eW3, eb3,
          dW1, db1, dg1, dbe1, dW2, db2, dg2, dbe2, dW3, db3,
          cb_ref,
          xhat_ref, r0, r1, r2, r3, c0, c1, c2, c3,
          es_ref, hi_ref, mid_ref, lo_ref):
    pid = pl.program_id(0)
    res_refs = (r0, r1, r2, r3)
    ce_refs = (c0, c1, c2, c3)

    xb = x_ref[...]
    maskf = mask_ref[...]
    xb = jnp.where(maskf > 0.5, 0.0, xb)
    row = jax.lax.broadcasted_iota(jnp.int32, (BLK, 1), 0) + pid * BLK
    xb = jnp.where(row == 0, 0.0, xb)
    xb = xb + pos_ref[...]

    h = _nt(xb, eW1[...]) + eb1[...]
    h = _bn_relu(h, eg1[...], ebe1[...])
    h = _nt(h, eW2[...]) + eb2[...]
    h = _bn_relu(h, eg2[...], ebe2[...])
    ze = _nt(h, eW3[...]) + eb3[...]

    @pl.when(pid == 0)
    def _():
        eye = (jax.lax.broadcasted_iota(jnp.int32, (K, K), 0)
               == jax.lax.broadcasted_iota(jnp.int32, (K, K), 1)
               ).astype(jnp.float32)
        for m in range(M_BOOK):
            embsq_col = _colsq_fold(cb_ref[m])
            es_ref[m:m + 1, :] = jax.lax.dot_general(
                embsq_col, eye, (((0,), (0,)), ((), ())),
                preferred_element_type=jnp.float32, precision=_HI)
        cb = cb_ref[...]
        msk = jnp.uint32(0xFFFF0000)
        hi = jax.lax.bitcast_convert_type(
            jax.lax.bitcast_convert_type(cb, jnp.uint32) & msk, jnp.float32)
        r1 = cb - hi
        mid = jax.lax.bitcast_convert_type(
            jax.lax.bitcast_convert_type(r1, jnp.uint32) & msk, jnp.float32)
        hi_ref[...] = hi.astype(jnp.bfloat16)
        mid_ref[...] = mid.astype(jnp.bfloat16)
        lo_ref[...] = (r1 - mid).astype(jnp.bfloat16)

    iota = jax.lax.broadcasted_iota(jnp.int32, (BLK, K), 1)
    cur = ze
    zq = None
    for m in range(M_BOOK):
        emb = cb_ref[m]
        rowsq = _rowsq(cur)
        embsq = es_ref[m:m + 1, :]
        score = _nt(cur, emb)
        dist = (rowsq + embsq) - 2.0 * score
        dmin = jnp.min(dist, axis=1, keepdims=True)
        nearest = jnp.min(jnp.where(dist == dmin, iota, K),
                          axis=1, keepdims=True)
        onehot = (iota == nearest).astype(jnp.bfloat16)
        def _sel(part_ref):
            return jax.lax.dot_general(
                onehot, part_ref[m], (((1,), (0,)), ((), ())),
                preferred_element_type=jnp.float32)
        ce = (_sel(hi_ref) + _sel(mid_ref)) + _sel(lo_ref)
        res_refs[m][...] = cur
        ce_refs[m][...] = ce
        zq = ce if zq is None else zq + ce
        cur = cur - ce

    di = ze + (zq - ze)
    h = _nt(di, dW1[...]) + db1[...]
    h = _bn_relu(h, dg1[...], dbe1[...])
    h = _nt(h, dW2[...]) + db2[...]
    h = _bn_relu(h, dg2[...], dbe2[...])
    xhat_ref[...] = _nt(h, dW3[...]) + db3[...]


def kernel(x, enc_W1, enc_b1, enc_g1, enc_be1, enc_W2, enc_b2, enc_g2,
           enc_be2, enc_W3, enc_b3, dec_W1, dec_b1, dec_g1, dec_be1,
           dec_W2, dec_b2, dec_g2, dec_be2, dec_W3, dec_b3,
           codebooks, pos, mask):
    row = lambda v: v.reshape(1, -1).astype(jnp.float32)
    maskf = row(mask)

    grid = BATCH // BLK
    full = lambda a: pl.BlockSpec(a.shape, lambda i: (0,) * a.ndim)
    batch_spec = lambda w: pl.BlockSpec((BLK, w), lambda i: (i, 0))

    args = (x, maskf, pos,
            enc_W1, row(enc_b1), row(enc_g1), row(enc_be1),
            enc_W2, row(enc_b2), row(enc_g2), row(enc_be2),
            enc_W3, row(enc_b3),
            dec_W1, row(dec_b1), row(dec_g1), row(dec_be1),
            dec_W2, row(dec_b2), row(dec_g2), row(dec_be2),
            dec_W3, row(dec_b3),
            codebooks)
    in_specs = [batch_spec(INPUT_DIM)] + [full(a) for a in args[1:]]

    out_shapes = ([jax.ShapeDtypeStruct((BATCH, INPUT_DIM), jnp.float32)]
                  + [jax.ShapeDtypeStruct((BATCH, DIM), jnp.float32)] * 8)
    out_specs = ([batch_spec(INPUT_DIM)] + [batch_spec(DIM)] * 8)

    outs = pl.pallas_call(
        _body,
        grid=(grid,),
        in_specs=in_specs,
        out_specs=out_specs,
        out_shape=out_shapes,
        scratch_shapes=[pltpu.VMEM((8, K), jnp.float32),
                        pltpu.VMEM((M_BOOK, K, DIM), jnp.bfloat16),
                        pltpu.VMEM((M_BOOK, K, DIM), jnp.bfloat16),
                        pltpu.VMEM((M_BOOK, K, DIM), jnp.bfloat16)],
    )(*args)

    x_hat = outs[0]
    res_list = tuple(outs[1:5])
    ce_list = tuple(outs[5:9])
    return (x_hat, res_list, ce_list)
